# baseline (speedup 1.0000x reference)
import functools

import jax
import jax.numpy as jnp
from jax import lax
from jax.experimental import pallas as pl
from jax.experimental.pallas import tpu as pltpu
from jax.experimental.pallas import tpu_sc as plsc

B, L, D, V = 4096, 50, 128, 1000000
N = B * L
EPS = 1e-12

NC, NS = 2, 16
NW = NC * NS
PER_W = N // NW
CHUNK = 128
NCHUNK = PER_W // CHUNK
ROWS_PER_W = PER_W // CHUNK


def _gather_body(idx_hbm, table_hbm, out_hbm, idx_v, rows_v, sem):
    wid = lax.axis_index("s") * NC + lax.axis_index("c")
    base = wid * PER_W
    pltpu.sync_copy(idx_hbm.at[pl.ds(base, PER_W)], idx_v)

    def body(j, carry):
        pltpu.async_copy(
            table_hbm.at[idx_v.at[pl.ds(j * CHUNK, CHUNK)]], rows_v, sem
        ).wait()
        pltpu.sync_copy(rows_v, out_hbm.at[pl.ds(base + j * CHUNK, CHUNK)])
        return carry

    lax.fori_loop(0, NCHUNK, body, 0)


@functools.cache
def _make_gather():
    return pl.kernel(
        _gather_body,
        mesh=plsc.VectorSubcoreMesh(core_axis_name="c", subcore_axis_name="s"),
        out_type=jax.ShapeDtypeStruct((N, D), jnp.float32),
        scratch_types=[
            pltpu.VMEM((PER_W,), jnp.int32),
            pltpu.VMEM((CHUNK, D), jnp.float32),
            pltpu.SemaphoreType.DMA,
        ],
    )


TB = 2048


def _tc_body(g_ref, bx_ref, sc_ref, wb_ref, bb_ref, ws_ref, bs_ref, gm_ref,
             bt_ref, o_ref):
    emb = g_ref[...]
    bx = bx_ref[...]
    wb = wb_ref[...]
    for k in range(4):
        emb += bx[:, k:k + 1] * wb[k:k + 1, :]
    emb += sc_ref[...] * ws_ref[...]
    emb += bb_ref[...] + bs_ref[...]
    mu = jnp.mean(emb, axis=-1, keepdims=True)
    dev = emb - mu
    var = jnp.mean(dev * dev, axis=-1, keepdims=True)
    o_ref[...] = dev * lax.rsqrt(var + EPS) * gm_ref[...] + bt_ref[...]


def _tc_call(gathered, bx, sc, wb, bb, ws, bs, gm, bt):
    grid = (N // TB,)
    return pl.pallas_call(
        _tc_body,
        grid=grid,
        in_specs=[
            pl.BlockSpec((TB, D), lambda i: (i, 0)),
            pl.BlockSpec((TB, 4), lambda i: (i, 0)),
            pl.BlockSpec((TB, 1), lambda i: (i, 0)),
            pl.BlockSpec((4, D), lambda i: (0, 0)),
            pl.BlockSpec((1, D), lambda i: (0, 0)),
            pl.BlockSpec((1, D), lambda i: (0, 0)),
            pl.BlockSpec((1, D), lambda i: (0, 0)),
            pl.BlockSpec((1, D), lambda i: (0, 0)),
            pl.BlockSpec((1, D), lambda i: (0, 0)),
        ],
        out_specs=pl.BlockSpec((TB, D), lambda i: (i, 0)),
        out_shape=jax.ShapeDtypeStruct((N, D), jnp.float32),
    )(gathered, bx, sc, wb, bb, ws, bs, gm, bt)


def kernel(categories, boxes, scores, table, W_box, b_box, W_score, b_score,
           gamma, beta):
    idx = categories.reshape(N).astype(jnp.int32)
    gathered = _make_gather()(idx, table)
    out = _tc_call(
        gathered,
        boxes.reshape(N, 4),
        scores.reshape(N, 1),
        W_box,
        b_box.reshape(1, D),
        W_score.reshape(1, D),
        b_score.reshape(1, D),
        gamma.reshape(1, D),
        beta.reshape(1, D),
    )
    return out.reshape(B, L, D)
